# baseline (speedup 1.0000x reference)
import functools

import jax
import jax.numpy as jnp
from jax import lax
from jax.experimental import pallas as pl
from jax.experimental.pallas import tpu as pltpu
from jax.experimental.pallas import tpu_sc as plsc

BATCH = 16384
DIM = 64
NC = 2
NS = 16
NW = NC * NS
BPW = BATCH // NW
CHUNK = 128
NCHUNK = BPW // CHUNK
L = 16

_mesh = plsc.VectorSubcoreMesh(core_axis_name="c", subcore_axis_name="s")

_GATHER_DNUMS = lax.GatherDimensionNumbers(
    offset_dims=(), collapsed_slice_dims=(0,), start_index_map=(0,))


def _permute(x, idx):
    return lax.gather(x, idx[:, None], _GATHER_DNUMS, slice_sizes=(1,),
                      mode=lax.GatherScatterMode.PROMISE_IN_BOUNDS)


def _allsum(x, lane):
    for m in (8, 4, 2, 1):
        x = x + _permute(x, lane ^ m)
    return x


@functools.partial(
    pl.kernel,
    mesh=_mesh,
    out_type=jax.ShapeDtypeStruct((BATCH,), jnp.float32),
    compiler_params=pltpu.CompilerParams(use_tc_tiling_on_sc=False),
    scratch_types=[
        pltpu.VMEM((NCHUNK, CHUNK), jnp.int32),
        pltpu.VMEM((NCHUNK, CHUNK), jnp.int32),
        pltpu.VMEM((NCHUNK, CHUNK), jnp.int32),
        pltpu.VMEM((CHUNK, DIM), jnp.float32),
        pltpu.VMEM((CHUNK, DIM), jnp.float32),
        pltpu.VMEM((CHUNK, DIM), jnp.float32),
        pltpu.VMEM((CHUNK, DIM), jnp.float32),
        pltpu.VMEM((CHUNK, DIM), jnp.float32),
        pltpu.VMEM((CHUNK, DIM), jnp.float32),
        pltpu.VMEM((BPW,), jnp.float32),
        pltpu.SemaphoreType.DMA,
    ],
)
def _complex_sc(hs_hbm, rs_hbm, ts_hbm, ent_re_hbm, ent_im_hbm,
                rel_re_hbm, rel_im_hbm, out_hbm,
                hs_v, rs_v, ts_v, reh, imh, ret, imt, rre, rim, out_v, sem):
    wid = lax.axis_index("s") * NC + lax.axis_index("c")
    base = wid * BPW
    for c in range(NCHUNK):
        off = base + c * CHUNK
        pltpu.sync_copy(hs_hbm.at[pl.ds(off, CHUNK)], hs_v.at[c])
        pltpu.sync_copy(rs_hbm.at[pl.ds(off, CHUNK)], rs_v.at[c])
        pltpu.sync_copy(ts_hbm.at[pl.ds(off, CHUNK)], ts_v.at[c])

    for c in range(NCHUNK):
        copies = [
            pltpu.async_copy(ent_re_hbm.at[hs_v.at[c]], reh, sem),
            pltpu.async_copy(ent_im_hbm.at[hs_v.at[c]], imh, sem),
            pltpu.async_copy(ent_re_hbm.at[ts_v.at[c]], ret, sem),
            pltpu.async_copy(ent_im_hbm.at[ts_v.at[c]], imt, sem),
            pltpu.async_copy(rel_re_hbm.at[rs_v.at[c]], rre, sem),
            pltpu.async_copy(rel_im_hbm.at[rs_v.at[c]], rim, sem),
        ]
        for cp in copies:
            cp.wait()

        def group(g, _, c=c):
            lane = lax.iota(jnp.int32, L)
            scores = jnp.zeros((L,), jnp.float32)
            for k in range(L):
                i = g * L + k
                acc = jnp.zeros((L,), jnp.float32)
                for j in range(DIM // L):
                    sl = pl.ds(j * L, L)
                    a = reh[i, sl]
                    b = imh[i, sl]
                    u = ret[i, sl]
                    v = imt[i, sl]
                    p = rre[i, sl]
                    q = rim[i, sl]
                    acc = acc + p * (a * u + b * v) + q * (a * v - b * u)
                scores = jnp.where(lane == k, _allsum(acc, lane), scores)
            out_v[pl.ds(c * CHUNK + g * L, L)] = scores
            return 0

        lax.fori_loop(0, CHUNK // L, group, 0)

    pltpu.sync_copy(out_v, out_hbm.at[pl.ds(base, BPW)])


def kernel(hs, rs, ts, ent_re, ent_im, rel_re, rel_im):
    return _complex_sc(hs, rs, ts, ent_re, ent_im, rel_re, rel_im)
